# baseline (speedup 1.0000x reference)
import jax
import jax.numpy as jnp
from jax import lax
from jax.experimental import pallas as pl
from jax.experimental.pallas import tpu as pltpu
from jax.experimental.pallas import tpu_sc as plsc

IOU_T = 0.5
SCORE_T = 0.5
MAXDET = 100
NEG_INF = float("-inf")

NIMG = 8
NBOX = 20000
NCLS = 80
NWORK = 32
WPI = NWORK // NIMG
CHUNK = NBOX // WPI
QUOTA = 512
TIER = WPI * QUOTA
TIER_TOTAL = 1024
T_FIX = float((1.0 - TIER_TOTAL / NBOX) ** (1.0 / NCLS))
SELPAD = 1024
ROWS_W = SELPAD // NWORK


def _conf_kernel(cls_ref, conf_ref, cid_ref):
    c = cls_ref[...]
    m = jnp.max(c, axis=-1)
    conf_ref[...] = m
    i3 = lax.broadcasted_iota(jnp.int32, c.shape, 2)
    cid_ref[...] = jnp.min(jnp.where(c == m[..., None], i3, c.shape[-1]),
                           axis=-1)


def _sc_compact_kernel(conf_hbm, box_hbm, cid_hbm,
                       sco_hbm, y0o_hbm, x0o_hbm, y1o_hbm, x1o_hbm,
                       cido_hbm, idxo_hbm, cnt_hbm,
                       conf_v, box_v, cid_v, sbuf, ibuf,
                       y0b, x0b, y1b, x1b, cidb, igb, cbuf):
    wid = lax.axis_index("s") * 2 + lax.axis_index("c")
    img = wid // WPI
    wq = wid % WPI
    base = img * NBOX + wq * CHUNK

    pltpu.sync_copy(conf_hbm.at[pl.ds(base, CHUNK)], conf_v.at[pl.ds(0, CHUNK)])
    pltpu.sync_copy(box_hbm.at[pl.ds(base * 4, CHUNK * 4)], box_v)
    pltpu.sync_copy(cid_hbm.at[pl.ds(base, CHUNK)], cid_v.at[pl.ds(0, CHUNK)])

    lane = lax.broadcasted_iota(jnp.int32, (16,), 0)

    def pre(j, _):
        sbuf[pl.ds(j * 16, 16)] = jnp.full((16,), NEG_INF, jnp.float32)
        ibuf[pl.ds(j * 16, 16)] = jnp.zeros((16,), jnp.int32)
        return 0
    lax.fori_loop(0, (QUOTA + 16) // 16, pre, 0)

    def scan(i, off):
        b = i * 16
        v = conf_v[pl.ds(b, 16)]
        li = lane + b
        m = (li < CHUNK) & (v >= T_FIX)
        incl = plsc.cumsum(m.astype(jnp.int32))
        dest = off + incl - 1
        plsc.store_scatter(sbuf, [dest], v, mask=m)
        plsc.store_scatter(ibuf, [dest], li, mask=m)
        return off + jnp.max(incl)
    nscan = (CHUNK + 15) // 16
    cnt = lax.fori_loop(0, nscan, scan, jnp.int32(0))

    def clean(j, _):
        b = j * 16
        sl = lane + b
        s = sbuf[pl.ds(b, 16)]
        iv = ibuf[pl.ds(b, 16)]
        sbuf[pl.ds(b, 16)] = jnp.where(sl < cnt, s, NEG_INF)
        ibuf[pl.ds(b, 16)] = jnp.where(sl < cnt, iv, 0)
        return 0
    lax.fori_loop(0, QUOTA // 16, clean, 0)

    def gath(j, _):
        b = j * 16
        il = ibuf[pl.ds(b, 16)]
        i4 = il * 4
        y0b[pl.ds(b, 16)] = plsc.load_gather(box_v, [i4])
        x0b[pl.ds(b, 16)] = plsc.load_gather(box_v, [i4 + 1])
        y1b[pl.ds(b, 16)] = plsc.load_gather(box_v, [i4 + 2])
        x1b[pl.ds(b, 16)] = plsc.load_gather(box_v, [i4 + 3])
        cidb[pl.ds(b, 16)] = plsc.load_gather(cid_v, [il])
        igb[pl.ds(b, 16)] = il + wq * CHUNK
        return 0
    lax.fori_loop(0, QUOTA // 16, gath, 0)

    obase = img * TIER + wq * QUOTA
    pltpu.sync_copy(sbuf.at[pl.ds(0, QUOTA)], sco_hbm.at[pl.ds(obase, QUOTA)])
    pltpu.sync_copy(y0b, y0o_hbm.at[pl.ds(obase, QUOTA)])
    pltpu.sync_copy(x0b, x0o_hbm.at[pl.ds(obase, QUOTA)])
    pltpu.sync_copy(y1b, y1o_hbm.at[pl.ds(obase, QUOTA)])
    pltpu.sync_copy(x1b, x1o_hbm.at[pl.ds(obase, QUOTA)])
    pltpu.sync_copy(cidb, cido_hbm.at[pl.ds(obase, QUOTA)])
    pltpu.sync_copy(igb, idxo_hbm.at[pl.ds(obase, QUOTA)])
    cbuf[...] = jnp.zeros((16,), jnp.int32) + cnt
    pltpu.sync_copy(cbuf, cnt_hbm.at[pl.ds(wid * 16, 16)])


def _nms_kernel(conf_ref, y0_ref, x0_ref, y1_ref, x1_ref, cid_ref,
                sc_ref, y0c_ref, x0c_ref, y1c_ref, x1c_ref,
                cidc_ref, idxc_ref, wcnt_ref,
                selo_ref, confo_ref, cido_ref,
                by0o_ref, bx0o_ref, by1o_ref, bx1o_ref, nvo_ref,
                work_ref):
    acc_shape = (NIMG, 128)
    lane = lax.broadcasted_iota(jnp.int32, acc_shape, 1)
    conf0 = conf_ref[:, 0:1]
    y00 = y0_ref[:, 0:1]; x00 = x0_ref[:, 0:1]
    y10 = y1_ref[:, 0:1]; x10 = x1_ref[:, 0:1]
    cid0 = cid_ref[:, 0:1]
    zf = jnp.zeros(acc_shape, jnp.float32)
    zi = jnp.zeros(acc_shape, jnp.int32)
    init = (zi, zf, zi, zf, zf, zf, zf, zi)

    def round_core(w, idx_arr, y0, x0, y1, x1, cidv, i, state):
        sel_a, conf_a, cid_a, b0_a, b1_a, b2_a, b3_a, nv_a = state
        m = jnp.max(w, axis=1, keepdims=True)
        valid = m > NEG_INF
        eq = w == m
        besti = jnp.min(jnp.where(eq, idx_arr, NBOX), axis=1, keepdims=True)
        bm = eq & (idx_arr == besti)
        best = jnp.where(valid, besti, 0)

        def gthf(a, fallback):
            g = jnp.sum(jnp.where(bm, a, 0.0), axis=1, keepdims=True)
            return jnp.where(valid, g, fallback)

        by0 = gthf(y0, y00); bx0 = gthf(x0, x00)
        by1 = gthf(y1, y10); bx1 = gthf(x1, x10)
        bcid = jnp.sum(jnp.where(bm, cidv, 0), axis=1, keepdims=True)
        bcid = jnp.where(valid, bcid, cid0)
        bymin = jnp.minimum(by0, by1); bymax = jnp.maximum(by0, by1)
        bxmin = jnp.minimum(bx0, bx1); bxmax = jnp.maximum(bx0, bx1)
        ymin = jnp.minimum(y0, y1); ymax = jnp.maximum(y0, y1)
        xmin = jnp.minimum(x0, x1); xmax = jnp.maximum(x0, x1)
        inter_h = jnp.maximum(0.0, jnp.minimum(bymax, ymax)
                              - jnp.maximum(bymin, ymin))
        inter_w = jnp.maximum(0.0, jnp.minimum(bxmax, xmax)
                              - jnp.maximum(bxmin, xmin))
        inter = inter_h * inter_w
        area1 = (bymax - bymin) * (bxmax - bxmin)
        area = (ymax - ymin) * (xmax - xmin)
        union = area1 + area - inter
        iou = jnp.where(union > 0, inter / union, 0.0)
        w_new = jnp.where(((iou > IOU_T) & valid) | bm, NEG_INF, w)

        hit = lane == i
        confp = jnp.where(valid, m, conf0)
        sel_a = jnp.where(hit, jnp.broadcast_to(best, acc_shape), sel_a)
        conf_a = jnp.where(hit, jnp.broadcast_to(confp, acc_shape), conf_a)
        cid_a = jnp.where(hit, jnp.broadcast_to(bcid, acc_shape), cid_a)
        b0_a = jnp.where(hit, jnp.broadcast_to(by0, acc_shape), b0_a)
        b1_a = jnp.where(hit, jnp.broadcast_to(bx0, acc_shape), b1_a)
        b2_a = jnp.where(hit, jnp.broadcast_to(by1, acc_shape), b2_a)
        b3_a = jnp.where(hit, jnp.broadcast_to(bx1, acc_shape), b3_a)
        nv_a = nv_a + jnp.broadcast_to(valid.astype(jnp.int32), acc_shape)
        return (sel_a, conf_a, cid_a, b0_a, b1_a, b2_a, b3_a, nv_a), w_new

    idxc = idxc_ref[...]
    y0c = y0c_ref[...]; x0c = x0c_ref[...]
    y1c = y1c_ref[...]; x1c = x1c_ref[...]
    cidc = cidc_ref[...]

    def tier_body(i, st):
        w, state = st
        state, w = round_core(w, idxc, y0c, x0c, y1c, x1c, cidc, i, state)
        return w, state

    w0 = sc_ref[...]
    _, tier_state = lax.fori_loop(0, MAXDET, tier_body, (w0, init))
    nv_tier = tier_state[7][:, 0:1]

    conf = conf_ref[...]
    c_all = jnp.sum((conf >= SCORE_T).astype(jnp.int32), axis=1,
                    keepdims=True)
    c_tier = jnp.sum((conf >= T_FIX).astype(jnp.int32), axis=1,
                     keepdims=True)
    below = c_all > c_tier
    over = jnp.max(wcnt_ref[...], axis=1, keepdims=True) > QUOTA
    need_full = over | (below & (nv_tier < MAXDET))
    any_full = jnp.max(need_full.astype(jnp.int32))

    def full_path(_):
        cf = conf_ref[...]
        work_ref[...] = jnp.where(cf >= SCORE_T, cf, NEG_INF)
        iota = lax.broadcasted_iota(jnp.int32, (NIMG, NBOX), 1)
        y0 = y0_ref[...]; x0 = x0_ref[...]
        y1 = y1_ref[...]; x1 = x1_ref[...]
        cidv = cid_ref[...]

        def body(i, state):
            w = work_ref[...]
            state, w_new = round_core(w, iota, y0, x0, y1, x1, cidv, i, state)
            work_ref[...] = w_new
            return state

        return lax.fori_loop(0, MAXDET, body, init)

    def tier_path(_):
        return tier_state

    sel_a, conf_a, cid_a, b0_a, b1_a, b2_a, b3_a, nv_a = lax.cond(
        any_full > 0, full_path, tier_path, 0)
    selo_ref[...] = sel_a[:, :MAXDET]
    confo_ref[...] = conf_a[:, :MAXDET]
    cido_ref[...] = cid_a[:, :MAXDET]
    by0o_ref[...] = b0_a[:, :MAXDET]
    bx0o_ref[...] = b1_a[:, :MAXDET]
    by1o_ref[...] = b2_a[:, :MAXDET]
    bx1o_ref[...] = b3_a[:, :MAXDET]
    nvo_ref[...] = nv_a[:, :1]


def _run_conf(classes):
    nimg, n, nc = classes.shape
    blk = 2048
    conf, cid = pl.pallas_call(
        _conf_kernel,
        grid=(pl.cdiv(n, blk),),
        in_specs=[pl.BlockSpec((nimg, blk, nc), lambda i: (0, i, 0))],
        out_specs=[pl.BlockSpec((nimg, blk), lambda i: (0, i)),
                   pl.BlockSpec((nimg, blk), lambda i: (0, i))],
        out_shape=[jax.ShapeDtypeStruct((nimg, n), jnp.float32),
                   jax.ShapeDtypeStruct((nimg, n), jnp.int32)],
    )(classes)
    return conf, cid


def _run_compact(conf, boxes, cid):
    BUF = CHUNK + 16
    f32 = jnp.float32
    i32 = jnp.int32
    outs = pl.kernel(
        _sc_compact_kernel,
        out_type=[jax.ShapeDtypeStruct((NIMG * TIER,), f32),
                  jax.ShapeDtypeStruct((NIMG * TIER,), f32),
                  jax.ShapeDtypeStruct((NIMG * TIER,), f32),
                  jax.ShapeDtypeStruct((NIMG * TIER,), f32),
                  jax.ShapeDtypeStruct((NIMG * TIER,), f32),
                  jax.ShapeDtypeStruct((NIMG * TIER,), i32),
                  jax.ShapeDtypeStruct((NIMG * TIER,), i32),
                  jax.ShapeDtypeStruct((NWORK * 16,), i32)],
        mesh=plsc.VectorSubcoreMesh(core_axis_name="c", subcore_axis_name="s"),
        compiler_params=pltpu.CompilerParams(needs_layout_passes=False),
        scratch_types=[pltpu.VMEM((BUF,), f32),
                       pltpu.VMEM((CHUNK * 4,), f32),
                       pltpu.VMEM((BUF,), i32),
                       pltpu.VMEM((BUF + 16,), f32),
                       pltpu.VMEM((BUF + 16,), i32),
                       pltpu.VMEM((QUOTA,), f32),
                       pltpu.VMEM((QUOTA,), f32),
                       pltpu.VMEM((QUOTA,), f32),
                       pltpu.VMEM((QUOTA,), f32),
                       pltpu.VMEM((QUOTA,), i32),
                       pltpu.VMEM((QUOTA,), i32),
                       pltpu.VMEM((16,), i32)],
    )(conf.reshape(-1), boxes.reshape(-1), cid.reshape(-1))
    sco, y0o, x0o, y1o, x1o, cido, idxo, cnts = outs
    shp = (NIMG, TIER)
    return (sco.reshape(shp), y0o.reshape(shp), x0o.reshape(shp),
            y1o.reshape(shp), x1o.reshape(shp), cido.reshape(shp),
            idxo.reshape(shp), cnts.reshape(NWORK, 16))


def kernel(boxes, classes):
    conf, cid = _run_conf(classes)
    y0 = boxes[:, :, 0]; x0 = boxes[:, :, 1]
    y1 = boxes[:, :, 2]; x1 = boxes[:, :, 3]
    sc, y0c, x0c, y1c, x1c, cidc, idxc, cnts = _run_compact(conf, boxes, cid)
    wcnt = cnts.reshape(NIMG, WPI * 16)

    outs = pl.pallas_call(
        _nms_kernel,
        out_shape=[jax.ShapeDtypeStruct((NIMG, MAXDET), jnp.int32),
                   jax.ShapeDtypeStruct((NIMG, MAXDET), jnp.float32),
                   jax.ShapeDtypeStruct((NIMG, MAXDET), jnp.int32),
                   jax.ShapeDtypeStruct((NIMG, MAXDET), jnp.float32),
                   jax.ShapeDtypeStruct((NIMG, MAXDET), jnp.float32),
                   jax.ShapeDtypeStruct((NIMG, MAXDET), jnp.float32),
                   jax.ShapeDtypeStruct((NIMG, MAXDET), jnp.float32),
                   jax.ShapeDtypeStruct((NIMG, 1), jnp.int32)],
        scratch_shapes=[pltpu.VMEM((NIMG, NBOX), jnp.float32)],
    )(conf, y0, x0, y1, x1, cid,
      sc, y0c, x0c, y1c, x1c, cidc, idxc, wcnt)
    sel, confp, cidp, by0, bx0, by1, bx1, nv = outs
    box_prediction = jnp.stack([by0, bx0, by1, bx1], axis=-1)
    return box_prediction, confp, cidp, nv[:, 0]


# speedup vs baseline: 2.6291x; 2.6291x over previous
import jax
import jax.numpy as jnp
from jax import lax
from jax.experimental import pallas as pl
from jax.experimental.pallas import tpu as pltpu
from jax.experimental.pallas import tpu_sc as plsc

IOU_T = 0.5
SCORE_T = 0.5
MAXDET = 100
NEG_INF = float("-inf")

NIMG = 8
NBOX = 20000
NCLS = 80
NWORK = 32
WPI = NWORK // NIMG
CHUNK = NBOX // WPI
QUOTA = 512
TIER = WPI * QUOTA
TIER_TOTAL = 1024
T_FIX = float((1.0 - TIER_TOTAL / NBOX) ** (1.0 / NCLS))
SELPAD = 1024
ROWS_W = SELPAD // NWORK


def _conf_kernel(cls_ref, conf_ref, cid_ref):
    c = cls_ref[...]
    m = jnp.max(c, axis=1)
    conf_ref[...] = m
    i3 = lax.broadcasted_iota(jnp.int32, c.shape, 1)
    cid_ref[...] = jnp.min(jnp.where(c == m[:, None, :], i3, c.shape[1]),
                           axis=1)


def _sc_compact_kernel(conf_hbm, box_hbm, cid_hbm,
                       sco_hbm, y0o_hbm, x0o_hbm, y1o_hbm, x1o_hbm,
                       cido_hbm, idxo_hbm, cnt_hbm,
                       conf_v, y0_v, x0_v, y1_v, x1_v, cid_v, sbuf, ibuf,
                       y0b, x0b, y1b, x1b, cidb, igb, cbuf):
    wid = lax.axis_index("s") * 2 + lax.axis_index("c")
    img = wid // WPI
    wq = wid % WPI
    base = img * NBOX + wq * CHUNK
    bbase = img * (NBOX * 4) + wq * CHUNK

    pltpu.sync_copy(conf_hbm.at[pl.ds(base, CHUNK)], conf_v.at[pl.ds(0, CHUNK)])
    pltpu.sync_copy(box_hbm.at[pl.ds(bbase, CHUNK)], y0_v.at[pl.ds(0, CHUNK)])
    pltpu.sync_copy(box_hbm.at[pl.ds(bbase + NBOX, CHUNK)],
                    x0_v.at[pl.ds(0, CHUNK)])
    pltpu.sync_copy(box_hbm.at[pl.ds(bbase + 2 * NBOX, CHUNK)],
                    y1_v.at[pl.ds(0, CHUNK)])
    pltpu.sync_copy(box_hbm.at[pl.ds(bbase + 3 * NBOX, CHUNK)],
                    x1_v.at[pl.ds(0, CHUNK)])
    pltpu.sync_copy(cid_hbm.at[pl.ds(base, CHUNK)], cid_v.at[pl.ds(0, CHUNK)])

    lane = lax.broadcasted_iota(jnp.int32, (16,), 0)

    def pre(j, _):
        sbuf[pl.ds(j * 16, 16)] = jnp.full((16,), NEG_INF, jnp.float32)
        ibuf[pl.ds(j * 16, 16)] = jnp.zeros((16,), jnp.int32)
        return 0
    lax.fori_loop(0, (QUOTA + 16) // 16, pre, 0)

    def scan(i, off):
        b = i * 16
        v = conf_v[pl.ds(b, 16)]
        li = lane + b
        m = (li < CHUNK) & (v >= T_FIX)
        incl = plsc.cumsum(m.astype(jnp.int32))
        dest = off + incl - 1
        plsc.store_scatter(sbuf, [dest], v, mask=m)
        plsc.store_scatter(ibuf, [dest], li, mask=m)
        return off + jnp.max(incl)
    nscan = (CHUNK + 15) // 16
    cnt = lax.fori_loop(0, nscan, scan, jnp.int32(0))

    def clean(j, _):
        b = j * 16
        sl = lane + b
        s = sbuf[pl.ds(b, 16)]
        iv = ibuf[pl.ds(b, 16)]
        sbuf[pl.ds(b, 16)] = jnp.where(sl < cnt, s, NEG_INF)
        ibuf[pl.ds(b, 16)] = jnp.where(sl < cnt, iv, 0)
        return 0
    lax.fori_loop(0, QUOTA // 16, clean, 0)

    def gath(j, _):
        b = j * 16
        il = ibuf[pl.ds(b, 16)]
        y0b[pl.ds(b, 16)] = plsc.load_gather(y0_v, [il])
        x0b[pl.ds(b, 16)] = plsc.load_gather(x0_v, [il])
        y1b[pl.ds(b, 16)] = plsc.load_gather(y1_v, [il])
        x1b[pl.ds(b, 16)] = plsc.load_gather(x1_v, [il])
        cidb[pl.ds(b, 16)] = plsc.load_gather(cid_v, [il])
        igb[pl.ds(b, 16)] = il + wq * CHUNK
        return 0
    lax.fori_loop(0, QUOTA // 16, gath, 0)

    obase = img * TIER + wq * QUOTA
    pltpu.sync_copy(sbuf.at[pl.ds(0, QUOTA)], sco_hbm.at[pl.ds(obase, QUOTA)])
    pltpu.sync_copy(y0b, y0o_hbm.at[pl.ds(obase, QUOTA)])
    pltpu.sync_copy(x0b, x0o_hbm.at[pl.ds(obase, QUOTA)])
    pltpu.sync_copy(y1b, y1o_hbm.at[pl.ds(obase, QUOTA)])
    pltpu.sync_copy(x1b, x1o_hbm.at[pl.ds(obase, QUOTA)])
    pltpu.sync_copy(cidb, cido_hbm.at[pl.ds(obase, QUOTA)])
    pltpu.sync_copy(igb, idxo_hbm.at[pl.ds(obase, QUOTA)])
    cbuf[...] = jnp.zeros((16,), jnp.int32) + cnt
    pltpu.sync_copy(cbuf, cnt_hbm.at[pl.ds(wid * 16, 16)])


def _nms_kernel(conf_ref, bx_ref, cid_ref,
                sc_ref, y0c_ref, x0c_ref, y1c_ref, x1c_ref,
                cidc_ref, idxc_ref, wcnt_ref,
                selo_ref, confo_ref, cido_ref,
                by0o_ref, bx0o_ref, by1o_ref, bx1o_ref, nvo_ref,
                work_ref):
    acc_shape = (NIMG, 128)
    lane = lax.broadcasted_iota(jnp.int32, acc_shape, 1)
    conf0 = conf_ref[:, 0:1]
    b00 = bx_ref[:, :, 0:1]
    y00 = b00[:, 0]; x00 = b00[:, 1]
    y10 = b00[:, 2]; x10 = b00[:, 3]
    cid0 = cid_ref[:, 0:1]
    zf = jnp.zeros(acc_shape, jnp.float32)
    zi = jnp.zeros(acc_shape, jnp.int32)
    init = (zi, zf, zi, zf, zf, zf, zf, zi)

    def round_core(w, idx_arr, y0, x0, y1, x1, cidv, i, state):
        sel_a, conf_a, cid_a, b0_a, b1_a, b2_a, b3_a, nv_a = state
        m = jnp.max(w, axis=1, keepdims=True)
        valid = m > NEG_INF
        eq = w == m
        besti = jnp.min(jnp.where(eq, idx_arr, NBOX), axis=1, keepdims=True)
        bm = eq & (idx_arr == besti)
        best = jnp.where(valid, besti, 0)

        def gthf(a, fallback):
            g = jnp.sum(jnp.where(bm, a, 0.0), axis=1, keepdims=True)
            return jnp.where(valid, g, fallback)

        by0 = gthf(y0, y00); bx0 = gthf(x0, x00)
        by1 = gthf(y1, y10); bx1 = gthf(x1, x10)
        bcid = jnp.sum(jnp.where(bm, cidv, 0), axis=1, keepdims=True)
        bcid = jnp.where(valid, bcid, cid0)
        bymin = jnp.minimum(by0, by1); bymax = jnp.maximum(by0, by1)
        bxmin = jnp.minimum(bx0, bx1); bxmax = jnp.maximum(bx0, bx1)
        ymin = jnp.minimum(y0, y1); ymax = jnp.maximum(y0, y1)
        xmin = jnp.minimum(x0, x1); xmax = jnp.maximum(x0, x1)
        inter_h = jnp.maximum(0.0, jnp.minimum(bymax, ymax)
                              - jnp.maximum(bymin, ymin))
        inter_w = jnp.maximum(0.0, jnp.minimum(bxmax, xmax)
                              - jnp.maximum(bxmin, xmin))
        inter = inter_h * inter_w
        area1 = (bymax - bymin) * (bxmax - bxmin)
        area = (ymax - ymin) * (xmax - xmin)
        union = area1 + area - inter
        iou = jnp.where(union > 0, inter / union, 0.0)
        w_new = jnp.where(((iou > IOU_T) & valid) | bm, NEG_INF, w)

        hit = lane == i
        confp = jnp.where(valid, m, conf0)
        sel_a = jnp.where(hit, jnp.broadcast_to(best, acc_shape), sel_a)
        conf_a = jnp.where(hit, jnp.broadcast_to(confp, acc_shape), conf_a)
        cid_a = jnp.where(hit, jnp.broadcast_to(bcid, acc_shape), cid_a)
        b0_a = jnp.where(hit, jnp.broadcast_to(by0, acc_shape), b0_a)
        b1_a = jnp.where(hit, jnp.broadcast_to(bx0, acc_shape), b1_a)
        b2_a = jnp.where(hit, jnp.broadcast_to(by1, acc_shape), b2_a)
        b3_a = jnp.where(hit, jnp.broadcast_to(bx1, acc_shape), b3_a)
        nv_a = nv_a + jnp.broadcast_to(valid.astype(jnp.int32), acc_shape)
        return (sel_a, conf_a, cid_a, b0_a, b1_a, b2_a, b3_a, nv_a), w_new

    idxc = idxc_ref[...]
    y0c = y0c_ref[...]; x0c = x0c_ref[...]
    y1c = y1c_ref[...]; x1c = x1c_ref[...]
    cidc = cidc_ref[...]

    def tier_body(i, st):
        w, state = st
        state, w = round_core(w, idxc, y0c, x0c, y1c, x1c, cidc, i, state)
        return w, state

    w0 = sc_ref[...]
    _, tier_state = lax.fori_loop(0, MAXDET, tier_body, (w0, init))
    nv_tier = tier_state[7][:, 0:1]

    conf = conf_ref[...]
    c_all = jnp.sum((conf >= SCORE_T).astype(jnp.int32), axis=1,
                    keepdims=True)
    c_tier = jnp.sum((conf >= T_FIX).astype(jnp.int32), axis=1,
                     keepdims=True)
    below = c_all > c_tier
    over = jnp.max(wcnt_ref[...], axis=1, keepdims=True) > QUOTA
    need_full = over | (below & (nv_tier < MAXDET))
    any_full = jnp.max(need_full.astype(jnp.int32))

    def full_path(_):
        cf = conf_ref[...]
        work_ref[...] = jnp.where(cf >= SCORE_T, cf, NEG_INF)
        iota = lax.broadcasted_iota(jnp.int32, (NIMG, NBOX), 1)
        y0 = bx_ref[:, 0, :]; x0 = bx_ref[:, 1, :]
        y1 = bx_ref[:, 2, :]; x1 = bx_ref[:, 3, :]
        cidv = cid_ref[...]

        def body(i, state):
            w = work_ref[...]
            state, w_new = round_core(w, iota, y0, x0, y1, x1, cidv, i, state)
            work_ref[...] = w_new
            return state

        return lax.fori_loop(0, MAXDET, body, init)

    def tier_path(_):
        return tier_state

    sel_a, conf_a, cid_a, b0_a, b1_a, b2_a, b3_a, nv_a = lax.cond(
        any_full > 0, full_path, tier_path, 0)
    selo_ref[...] = sel_a[:, :MAXDET]
    confo_ref[...] = conf_a[:, :MAXDET]
    cido_ref[...] = cid_a[:, :MAXDET]
    by0o_ref[...] = b0_a[:, :MAXDET]
    bx0o_ref[...] = b1_a[:, :MAXDET]
    by1o_ref[...] = b2_a[:, :MAXDET]
    bx1o_ref[...] = b3_a[:, :MAXDET]
    nvo_ref[...] = nv_a[:, :1]


def _run_conf(classes):
    nimg, n, nc = classes.shape
    cls_t = jnp.transpose(classes, (0, 2, 1))
    blk = 2048
    conf, cid = pl.pallas_call(
        _conf_kernel,
        grid=(pl.cdiv(n, blk),),
        in_specs=[pl.BlockSpec((nimg, nc, blk), lambda i: (0, 0, i))],
        out_specs=[pl.BlockSpec((nimg, blk), lambda i: (0, i)),
                   pl.BlockSpec((nimg, blk), lambda i: (0, i))],
        out_shape=[jax.ShapeDtypeStruct((nimg, n), jnp.float32),
                   jax.ShapeDtypeStruct((nimg, n), jnp.int32)],
    )(cls_t)
    return conf, cid


def _run_compact(conf, bx_t, cid):
    BUF = CHUNK + 16
    f32 = jnp.float32
    i32 = jnp.int32
    outs = pl.kernel(
        _sc_compact_kernel,
        out_type=[jax.ShapeDtypeStruct((NIMG * TIER,), f32),
                  jax.ShapeDtypeStruct((NIMG * TIER,), f32),
                  jax.ShapeDtypeStruct((NIMG * TIER,), f32),
                  jax.ShapeDtypeStruct((NIMG * TIER,), f32),
                  jax.ShapeDtypeStruct((NIMG * TIER,), f32),
                  jax.ShapeDtypeStruct((NIMG * TIER,), i32),
                  jax.ShapeDtypeStruct((NIMG * TIER,), i32),
                  jax.ShapeDtypeStruct((NWORK * 16,), i32)],
        mesh=plsc.VectorSubcoreMesh(core_axis_name="c", subcore_axis_name="s"),
        compiler_params=pltpu.CompilerParams(needs_layout_passes=False),
        scratch_types=[pltpu.VMEM((BUF,), f32),
                       pltpu.VMEM((BUF,), f32),
                       pltpu.VMEM((BUF,), f32),
                       pltpu.VMEM((BUF,), f32),
                       pltpu.VMEM((BUF,), f32),
                       pltpu.VMEM((BUF,), i32),
                       pltpu.VMEM((BUF + 16,), f32),
                       pltpu.VMEM((BUF + 16,), i32),
                       pltpu.VMEM((QUOTA,), f32),
                       pltpu.VMEM((QUOTA,), f32),
                       pltpu.VMEM((QUOTA,), f32),
                       pltpu.VMEM((QUOTA,), f32),
                       pltpu.VMEM((QUOTA,), i32),
                       pltpu.VMEM((QUOTA,), i32),
                       pltpu.VMEM((16,), i32)],
    )(conf.reshape(-1), bx_t.reshape(-1), cid.reshape(-1))
    sco, y0o, x0o, y1o, x1o, cido, idxo, cnts = outs
    shp = (NIMG, TIER)
    return (sco.reshape(shp), y0o.reshape(shp), x0o.reshape(shp),
            y1o.reshape(shp), x1o.reshape(shp), cido.reshape(shp),
            idxo.reshape(shp), cnts.reshape(NWORK, 16))


def kernel(boxes, classes):
    conf, cid = _run_conf(classes)
    bx_t = jnp.transpose(boxes, (0, 2, 1))
    sc, y0c, x0c, y1c, x1c, cidc, idxc, cnts = _run_compact(conf, bx_t, cid)
    wcnt = cnts.reshape(NIMG, WPI * 16)

    outs = pl.pallas_call(
        _nms_kernel,
        out_shape=[jax.ShapeDtypeStruct((NIMG, MAXDET), jnp.int32),
                   jax.ShapeDtypeStruct((NIMG, MAXDET), jnp.float32),
                   jax.ShapeDtypeStruct((NIMG, MAXDET), jnp.int32),
                   jax.ShapeDtypeStruct((NIMG, MAXDET), jnp.float32),
                   jax.ShapeDtypeStruct((NIMG, MAXDET), jnp.float32),
                   jax.ShapeDtypeStruct((NIMG, MAXDET), jnp.float32),
                   jax.ShapeDtypeStruct((NIMG, MAXDET), jnp.float32),
                   jax.ShapeDtypeStruct((NIMG, 1), jnp.int32)],
        scratch_shapes=[pltpu.VMEM((NIMG, NBOX), jnp.float32)],
    )(conf, bx_t, cid,
      sc, y0c, x0c, y1c, x1c, cidc, idxc, wcnt)
    sel, confp, cidp, by0, bx0, by1, bx1, nv = outs
    box_prediction = jnp.stack([by0, bx0, by1, bx1], axis=-1)
    return box_prediction, confp, cidp, nv[:, 0]


# speedup vs baseline: 2.6308x; 1.0006x over previous
import jax
import jax.numpy as jnp
from jax import lax
from jax.experimental import pallas as pl
from jax.experimental.pallas import tpu as pltpu
from jax.experimental.pallas import tpu_sc as plsc

IOU_T = 0.5
SCORE_T = 0.5
MAXDET = 100
NEG_INF = float("-inf")

NIMG = 8
NBOX = 20000
NCLS = 80
NWORK = 32
WPI = NWORK // NIMG
CHUNK = NBOX // WPI
QUOTA = 512
TIER = WPI * QUOTA
TIER_TOTAL = 1024
T_FIX = float((1.0 - TIER_TOTAL / NBOX) ** (1.0 / NCLS))
SELPAD = 1024
ROWS_W = SELPAD // NWORK


def _conf_kernel(cls_ref, conf_ref, cid_ref):
    c = cls_ref[...]
    m = jnp.max(c, axis=1)
    conf_ref[...] = m
    i3 = lax.broadcasted_iota(jnp.int32, c.shape, 1)
    cid_ref[...] = jnp.min(jnp.where(c == m[:, None, :], i3, c.shape[1]),
                           axis=1)


def _sc_compact_kernel(conf_hbm, box_hbm, cid_hbm,
                       sco_hbm, y0o_hbm, x0o_hbm, y1o_hbm, x1o_hbm,
                       cido_hbm, idxo_hbm, cnt_hbm,
                       conf_v, y0_v, x0_v, y1_v, x1_v, cid_v, sbuf, ibuf,
                       y0b, x0b, y1b, x1b, cidb, igb, cbuf):
    wid = lax.axis_index("s") * 2 + lax.axis_index("c")
    img = wid // WPI
    wq = wid % WPI
    base = img * NBOX + wq * CHUNK
    bbase = img * (NBOX * 4) + wq * CHUNK

    pltpu.sync_copy(conf_hbm.at[pl.ds(base, CHUNK)], conf_v.at[pl.ds(0, CHUNK)])
    pltpu.sync_copy(box_hbm.at[pl.ds(bbase, CHUNK)], y0_v.at[pl.ds(0, CHUNK)])
    pltpu.sync_copy(box_hbm.at[pl.ds(bbase + NBOX, CHUNK)],
                    x0_v.at[pl.ds(0, CHUNK)])
    pltpu.sync_copy(box_hbm.at[pl.ds(bbase + 2 * NBOX, CHUNK)],
                    y1_v.at[pl.ds(0, CHUNK)])
    pltpu.sync_copy(box_hbm.at[pl.ds(bbase + 3 * NBOX, CHUNK)],
                    x1_v.at[pl.ds(0, CHUNK)])
    pltpu.sync_copy(cid_hbm.at[pl.ds(base, CHUNK)], cid_v.at[pl.ds(0, CHUNK)])

    lane = lax.broadcasted_iota(jnp.int32, (16,), 0)

    def pre(j, _):
        sbuf[pl.ds(j * 16, 16)] = jnp.full((16,), NEG_INF, jnp.float32)
        ibuf[pl.ds(j * 16, 16)] = jnp.zeros((16,), jnp.int32)
        return 0
    lax.fori_loop(0, (QUOTA + 16) // 16, pre, 0)

    def scan(i, off):
        b = i * 16
        v = conf_v[pl.ds(b, 16)]
        li = lane + b
        m = (li < CHUNK) & (v >= T_FIX)
        incl = plsc.cumsum(m.astype(jnp.int32))
        dest = off + incl - 1
        plsc.store_scatter(sbuf, [dest], v, mask=m)
        plsc.store_scatter(ibuf, [dest], li, mask=m)
        return off + jnp.max(incl)
    nscan = (CHUNK + 15) // 16
    cnt = lax.fori_loop(0, nscan, scan, jnp.int32(0))

    def clean(j, _):
        b = j * 16
        sl = lane + b
        s = sbuf[pl.ds(b, 16)]
        iv = ibuf[pl.ds(b, 16)]
        sbuf[pl.ds(b, 16)] = jnp.where(sl < cnt, s, NEG_INF)
        ibuf[pl.ds(b, 16)] = jnp.where(sl < cnt, iv, 0)
        return 0
    lax.fori_loop(0, QUOTA // 16, clean, 0)

    def gath(j, _):
        b = j * 16
        il = ibuf[pl.ds(b, 16)]
        y0b[pl.ds(b, 16)] = plsc.load_gather(y0_v, [il])
        x0b[pl.ds(b, 16)] = plsc.load_gather(x0_v, [il])
        y1b[pl.ds(b, 16)] = plsc.load_gather(y1_v, [il])
        x1b[pl.ds(b, 16)] = plsc.load_gather(x1_v, [il])
        cidb[pl.ds(b, 16)] = plsc.load_gather(cid_v, [il])
        igb[pl.ds(b, 16)] = il + wq * CHUNK
        return 0
    lax.fori_loop(0, QUOTA // 16, gath, 0)

    obase = img * TIER + wq * QUOTA
    pltpu.sync_copy(sbuf.at[pl.ds(0, QUOTA)], sco_hbm.at[pl.ds(obase, QUOTA)])
    pltpu.sync_copy(y0b, y0o_hbm.at[pl.ds(obase, QUOTA)])
    pltpu.sync_copy(x0b, x0o_hbm.at[pl.ds(obase, QUOTA)])
    pltpu.sync_copy(y1b, y1o_hbm.at[pl.ds(obase, QUOTA)])
    pltpu.sync_copy(x1b, x1o_hbm.at[pl.ds(obase, QUOTA)])
    pltpu.sync_copy(cidb, cido_hbm.at[pl.ds(obase, QUOTA)])
    pltpu.sync_copy(igb, idxo_hbm.at[pl.ds(obase, QUOTA)])
    cbuf[...] = jnp.zeros((16,), jnp.int32) + cnt
    pltpu.sync_copy(cbuf, cnt_hbm.at[pl.ds(wid * 16, 16)])


def _nms_kernel(conf_ref, bx_ref, cid_ref,
                sc_ref, y0c_ref, x0c_ref, y1c_ref, x1c_ref,
                cidc_ref, idxc_ref, wcnt_ref,
                selo_ref, confo_ref, cido_ref,
                by0o_ref, bx0o_ref, by1o_ref, bx1o_ref, nvo_ref,
                work_ref):
    acc_shape = (NIMG, 128)
    lane = lax.broadcasted_iota(jnp.int32, acc_shape, 1)
    conf0 = conf_ref[:, 0:1]
    b00 = bx_ref[:, :, 0:1]
    y00 = b00[:, 0]; x00 = b00[:, 1]
    y10 = b00[:, 2]; x10 = b00[:, 3]
    cid0 = cid_ref[:, 0:1]
    zf = jnp.zeros(acc_shape, jnp.float32)
    zi = jnp.zeros(acc_shape, jnp.int32)
    init = (zi, zf, zi, zf, zf, zf, zf, zi)

    def round_core(w, idx_arr, y0, x0, y1, x1, cidv,
                   ymin, ymax, xmin, xmax, area, i, state):
        sel_a, conf_a, cid_a, b0_a, b1_a, b2_a, b3_a, nv_a = state
        m = jnp.max(w, axis=1, keepdims=True)
        valid = m > NEG_INF
        eq = w == m
        besti = jnp.min(jnp.where(eq, idx_arr, NBOX), axis=1, keepdims=True)
        bm = eq & (idx_arr == besti)
        best = jnp.where(valid, besti, 0)

        def gthf(a, fallback):
            g = jnp.sum(jnp.where(bm, a, 0.0), axis=1, keepdims=True)
            return jnp.where(valid, g, fallback)

        by0 = gthf(y0, y00); bx0 = gthf(x0, x00)
        by1 = gthf(y1, y10); bx1 = gthf(x1, x10)
        bcid = jnp.sum(jnp.where(bm, cidv, 0), axis=1, keepdims=True)
        bcid = jnp.where(valid, bcid, cid0)
        bymin = jnp.minimum(by0, by1); bymax = jnp.maximum(by0, by1)
        bxmin = jnp.minimum(bx0, bx1); bxmax = jnp.maximum(bx0, bx1)
        inter_h = jnp.maximum(0.0, jnp.minimum(bymax, ymax)
                              - jnp.maximum(bymin, ymin))
        inter_w = jnp.maximum(0.0, jnp.minimum(bxmax, xmax)
                              - jnp.maximum(bxmin, xmin))
        inter = inter_h * inter_w
        area1 = (bymax - bymin) * (bxmax - bxmin)
        union = area1 + area - inter
        iou = jnp.where(union > 0, inter / union, 0.0)
        w_new = jnp.where(((iou > IOU_T) & valid) | bm, NEG_INF, w)

        hit = lane == i
        confp = jnp.where(valid, m, conf0)
        sel_a = jnp.where(hit, jnp.broadcast_to(best, acc_shape), sel_a)
        conf_a = jnp.where(hit, jnp.broadcast_to(confp, acc_shape), conf_a)
        cid_a = jnp.where(hit, jnp.broadcast_to(bcid, acc_shape), cid_a)
        b0_a = jnp.where(hit, jnp.broadcast_to(by0, acc_shape), b0_a)
        b1_a = jnp.where(hit, jnp.broadcast_to(bx0, acc_shape), b1_a)
        b2_a = jnp.where(hit, jnp.broadcast_to(by1, acc_shape), b2_a)
        b3_a = jnp.where(hit, jnp.broadcast_to(bx1, acc_shape), b3_a)
        nv_a = nv_a + jnp.broadcast_to(valid.astype(jnp.int32), acc_shape)
        return (sel_a, conf_a, cid_a, b0_a, b1_a, b2_a, b3_a, nv_a), w_new

    idxc = idxc_ref[...]
    y0c = y0c_ref[...]; x0c = x0c_ref[...]
    y1c = y1c_ref[...]; x1c = x1c_ref[...]
    cidc = cidc_ref[...]
    yminc = jnp.minimum(y0c, y1c); ymaxc = jnp.maximum(y0c, y1c)
    xminc = jnp.minimum(x0c, x1c); xmaxc = jnp.maximum(x0c, x1c)
    areac = (ymaxc - yminc) * (xmaxc - xminc)

    def tier_body(i, st):
        w, state = st
        state, w = round_core(w, idxc, y0c, x0c, y1c, x1c, cidc,
                              yminc, ymaxc, xminc, xmaxc, areac, i, state)
        return w, state

    w0 = sc_ref[...]
    _, tier_state = lax.fori_loop(0, MAXDET, tier_body, (w0, init))
    nv_tier = tier_state[7][:, 0:1]

    conf = conf_ref[...]
    c_all = jnp.sum((conf >= SCORE_T).astype(jnp.int32), axis=1,
                    keepdims=True)
    c_tier = jnp.sum((conf >= T_FIX).astype(jnp.int32), axis=1,
                     keepdims=True)
    below = c_all > c_tier
    over = jnp.max(wcnt_ref[...], axis=1, keepdims=True) > QUOTA
    need_full = over | (below & (nv_tier < MAXDET))
    any_full = jnp.max(need_full.astype(jnp.int32))

    def full_path(_):
        cf = conf_ref[...]
        work_ref[...] = jnp.where(cf >= SCORE_T, cf, NEG_INF)
        iota = lax.broadcasted_iota(jnp.int32, (NIMG, NBOX), 1)
        y0 = bx_ref[:, 0, :]; x0 = bx_ref[:, 1, :]
        y1 = bx_ref[:, 2, :]; x1 = bx_ref[:, 3, :]
        cidv = cid_ref[...]
        ymin = jnp.minimum(y0, y1); ymax = jnp.maximum(y0, y1)
        xmin = jnp.minimum(x0, x1); xmax = jnp.maximum(x0, x1)
        area = (ymax - ymin) * (xmax - xmin)

        def body(i, state):
            w = work_ref[...]
            state, w_new = round_core(w, iota, y0, x0, y1, x1, cidv,
                                      ymin, ymax, xmin, xmax, area, i, state)
            work_ref[...] = w_new
            return state

        return lax.fori_loop(0, MAXDET, body, init)

    def tier_path(_):
        return tier_state

    sel_a, conf_a, cid_a, b0_a, b1_a, b2_a, b3_a, nv_a = lax.cond(
        any_full > 0, full_path, tier_path, 0)
    selo_ref[...] = sel_a[:, :MAXDET]
    confo_ref[...] = conf_a[:, :MAXDET]
    cido_ref[...] = cid_a[:, :MAXDET]
    by0o_ref[...] = b0_a[:, :MAXDET]
    bx0o_ref[...] = b1_a[:, :MAXDET]
    by1o_ref[...] = b2_a[:, :MAXDET]
    bx1o_ref[...] = b3_a[:, :MAXDET]
    nvo_ref[...] = nv_a[:, :1]


def _run_conf(classes):
    nimg, n, nc = classes.shape
    cls_t = jnp.transpose(classes, (0, 2, 1))
    blk = 2048
    conf, cid = pl.pallas_call(
        _conf_kernel,
        grid=(pl.cdiv(n, blk),),
        in_specs=[pl.BlockSpec((nimg, nc, blk), lambda i: (0, 0, i))],
        out_specs=[pl.BlockSpec((nimg, blk), lambda i: (0, i)),
                   pl.BlockSpec((nimg, blk), lambda i: (0, i))],
        out_shape=[jax.ShapeDtypeStruct((nimg, n), jnp.float32),
                   jax.ShapeDtypeStruct((nimg, n), jnp.int32)],
    )(cls_t)
    return conf, cid


def _run_compact(conf, bx_t, cid):
    BUF = CHUNK + 16
    f32 = jnp.float32
    i32 = jnp.int32
    outs = pl.kernel(
        _sc_compact_kernel,
        out_type=[jax.ShapeDtypeStruct((NIMG * TIER,), f32),
                  jax.ShapeDtypeStruct((NIMG * TIER,), f32),
                  jax.ShapeDtypeStruct((NIMG * TIER,), f32),
                  jax.ShapeDtypeStruct((NIMG * TIER,), f32),
                  jax.ShapeDtypeStruct((NIMG * TIER,), f32),
                  jax.ShapeDtypeStruct((NIMG * TIER,), i32),
                  jax.ShapeDtypeStruct((NIMG * TIER,), i32),
                  jax.ShapeDtypeStruct((NWORK * 16,), i32)],
        mesh=plsc.VectorSubcoreMesh(core_axis_name="c", subcore_axis_name="s"),
        compiler_params=pltpu.CompilerParams(needs_layout_passes=False),
        scratch_types=[pltpu.VMEM((BUF,), f32),
                       pltpu.VMEM((BUF,), f32),
                       pltpu.VMEM((BUF,), f32),
                       pltpu.VMEM((BUF,), f32),
                       pltpu.VMEM((BUF,), f32),
                       pltpu.VMEM((BUF,), i32),
                       pltpu.VMEM((BUF + 16,), f32),
                       pltpu.VMEM((BUF + 16,), i32),
                       pltpu.VMEM((QUOTA,), f32),
                       pltpu.VMEM((QUOTA,), f32),
                       pltpu.VMEM((QUOTA,), f32),
                       pltpu.VMEM((QUOTA,), f32),
                       pltpu.VMEM((QUOTA,), i32),
                       pltpu.VMEM((QUOTA,), i32),
                       pltpu.VMEM((16,), i32)],
    )(conf.reshape(-1), bx_t.reshape(-1), cid.reshape(-1))
    sco, y0o, x0o, y1o, x1o, cido, idxo, cnts = outs
    shp = (NIMG, TIER)
    return (sco.reshape(shp), y0o.reshape(shp), x0o.reshape(shp),
            y1o.reshape(shp), x1o.reshape(shp), cido.reshape(shp),
            idxo.reshape(shp), cnts.reshape(NWORK, 16))


def kernel(boxes, classes):
    conf, cid = _run_conf(classes)
    bx_t = jnp.transpose(boxes, (0, 2, 1))
    sc, y0c, x0c, y1c, x1c, cidc, idxc, cnts = _run_compact(conf, bx_t, cid)
    wcnt = cnts.reshape(NIMG, WPI * 16)

    outs = pl.pallas_call(
        _nms_kernel,
        out_shape=[jax.ShapeDtypeStruct((NIMG, MAXDET), jnp.int32),
                   jax.ShapeDtypeStruct((NIMG, MAXDET), jnp.float32),
                   jax.ShapeDtypeStruct((NIMG, MAXDET), jnp.int32),
                   jax.ShapeDtypeStruct((NIMG, MAXDET), jnp.float32),
                   jax.ShapeDtypeStruct((NIMG, MAXDET), jnp.float32),
                   jax.ShapeDtypeStruct((NIMG, MAXDET), jnp.float32),
                   jax.ShapeDtypeStruct((NIMG, MAXDET), jnp.float32),
                   jax.ShapeDtypeStruct((NIMG, 1), jnp.int32)],
        scratch_shapes=[pltpu.VMEM((NIMG, NBOX), jnp.float32)],
    )(conf, bx_t, cid,
      sc, y0c, x0c, y1c, x1c, cidc, idxc, wcnt)
    sel, confp, cidp, by0, bx0, by1, bx1, nv = outs
    box_prediction = jnp.stack([by0, bx0, by1, bx1], axis=-1)
    return box_prediction, confp, cidp, nv[:, 0]


# speedup vs baseline: 2.7181x; 1.0332x over previous
import jax
import jax.numpy as jnp
from jax import lax
from jax.experimental import pallas as pl
from jax.experimental.pallas import tpu as pltpu
from jax.experimental.pallas import tpu_sc as plsc

IOU_T = 0.5
SCORE_T = 0.5
MAXDET = 100
NEG_INF = float("-inf")

NIMG = 8
NBOX = 20000
NCLS = 80
NWORK = 32
WPI = NWORK // NIMG
CHUNK = NBOX // WPI
QUOTA = 384
TIER = WPI * QUOTA
TIER_TOTAL = 1024
T_FIX = float((1.0 - TIER_TOTAL / NBOX) ** (1.0 / NCLS))
SELPAD = 1024
ROWS_W = SELPAD // NWORK


def _conf_kernel(cls_ref, conf_ref, cid_ref):
    c = cls_ref[...]
    m = jnp.max(c, axis=1)
    conf_ref[...] = m
    i3 = lax.broadcasted_iota(jnp.int32, c.shape, 1)
    cid_ref[...] = jnp.min(jnp.where(c == m[:, None, :], i3, c.shape[1]),
                           axis=1)


def _sc_compact_kernel(conf_hbm, box_hbm, cid_hbm,
                       sco_hbm, y0o_hbm, x0o_hbm, y1o_hbm, x1o_hbm,
                       cido_hbm, idxo_hbm, cnt_hbm,
                       conf_v, y0_v, x0_v, y1_v, x1_v, cid_v, sbuf, ibuf,
                       y0b, x0b, y1b, x1b, cidb, igb, cbuf):
    wid = lax.axis_index("s") * 2 + lax.axis_index("c")
    img = wid // WPI
    wq = wid % WPI
    base = img * NBOX + wq * CHUNK
    bbase = img * (NBOX * 4) + wq * CHUNK

    pltpu.sync_copy(conf_hbm.at[pl.ds(base, CHUNK)], conf_v.at[pl.ds(0, CHUNK)])
    pltpu.sync_copy(box_hbm.at[pl.ds(bbase, CHUNK)], y0_v.at[pl.ds(0, CHUNK)])
    pltpu.sync_copy(box_hbm.at[pl.ds(bbase + NBOX, CHUNK)],
                    x0_v.at[pl.ds(0, CHUNK)])
    pltpu.sync_copy(box_hbm.at[pl.ds(bbase + 2 * NBOX, CHUNK)],
                    y1_v.at[pl.ds(0, CHUNK)])
    pltpu.sync_copy(box_hbm.at[pl.ds(bbase + 3 * NBOX, CHUNK)],
                    x1_v.at[pl.ds(0, CHUNK)])
    pltpu.sync_copy(cid_hbm.at[pl.ds(base, CHUNK)], cid_v.at[pl.ds(0, CHUNK)])

    lane = lax.broadcasted_iota(jnp.int32, (16,), 0)

    def pre(j, _):
        sbuf[pl.ds(j * 16, 16)] = jnp.full((16,), NEG_INF, jnp.float32)
        ibuf[pl.ds(j * 16, 16)] = jnp.zeros((16,), jnp.int32)
        return 0
    lax.fori_loop(0, (QUOTA + 16) // 16, pre, 0)

    def scan(i, off):
        b = i * 16
        v = conf_v[pl.ds(b, 16)]
        li = lane + b
        m = (li < CHUNK) & (v >= T_FIX)
        incl = plsc.cumsum(m.astype(jnp.int32))
        dest = off + incl - 1
        plsc.store_scatter(sbuf, [dest], v, mask=m)
        plsc.store_scatter(ibuf, [dest], li, mask=m)
        return off + jnp.max(incl)
    nscan = (CHUNK + 15) // 16
    cnt = lax.fori_loop(0, nscan, scan, jnp.int32(0))

    def clean(j, _):
        b = j * 16
        sl = lane + b
        s = sbuf[pl.ds(b, 16)]
        iv = ibuf[pl.ds(b, 16)]
        sbuf[pl.ds(b, 16)] = jnp.where(sl < cnt, s, NEG_INF)
        ibuf[pl.ds(b, 16)] = jnp.where(sl < cnt, iv, 0)
        return 0
    lax.fori_loop(0, QUOTA // 16, clean, 0)

    def gath(j, _):
        b = j * 16
        il = ibuf[pl.ds(b, 16)]
        y0b[pl.ds(b, 16)] = plsc.load_gather(y0_v, [il])
        x0b[pl.ds(b, 16)] = plsc.load_gather(x0_v, [il])
        y1b[pl.ds(b, 16)] = plsc.load_gather(y1_v, [il])
        x1b[pl.ds(b, 16)] = plsc.load_gather(x1_v, [il])
        cidb[pl.ds(b, 16)] = plsc.load_gather(cid_v, [il])
        igb[pl.ds(b, 16)] = il + wq * CHUNK
        return 0
    lax.fori_loop(0, QUOTA // 16, gath, 0)

    obase = img * TIER + wq * QUOTA
    pltpu.sync_copy(sbuf.at[pl.ds(0, QUOTA)], sco_hbm.at[pl.ds(obase, QUOTA)])
    pltpu.sync_copy(y0b, y0o_hbm.at[pl.ds(obase, QUOTA)])
    pltpu.sync_copy(x0b, x0o_hbm.at[pl.ds(obase, QUOTA)])
    pltpu.sync_copy(y1b, y1o_hbm.at[pl.ds(obase, QUOTA)])
    pltpu.sync_copy(x1b, x1o_hbm.at[pl.ds(obase, QUOTA)])
    pltpu.sync_copy(cidb, cido_hbm.at[pl.ds(obase, QUOTA)])
    pltpu.sync_copy(igb, idxo_hbm.at[pl.ds(obase, QUOTA)])
    cbuf[...] = jnp.zeros((16,), jnp.int32) + cnt
    pltpu.sync_copy(cbuf, cnt_hbm.at[pl.ds(wid * 16, 16)])


def _nms_kernel(conf_ref, bx_ref, cid_ref,
                sc_ref, y0c_ref, x0c_ref, y1c_ref, x1c_ref,
                cidc_ref, idxc_ref, wcnt_ref,
                selo_ref, confo_ref, cido_ref,
                by0o_ref, bx0o_ref, by1o_ref, bx1o_ref, nvo_ref,
                work_ref):
    acc_shape = (NIMG, 128)
    lane = lax.broadcasted_iota(jnp.int32, acc_shape, 1)
    conf0 = conf_ref[:, 0:1]
    b00 = bx_ref[:, :, 0:1]
    y00 = b00[:, 0]; x00 = b00[:, 1]
    y10 = b00[:, 2]; x10 = b00[:, 3]
    cid0 = cid_ref[:, 0:1]
    zf = jnp.zeros(acc_shape, jnp.float32)
    zi = jnp.zeros(acc_shape, jnp.int32)
    init = (zi, zf, zi, zf, zf, zf, zf, zi)

    def round_core(w, idx_arr, y0, x0, y1, x1, cidv,
                   ymin, ymax, xmin, xmax, area, i, state):
        sel_a, conf_a, cid_a, b0_a, b1_a, b2_a, b3_a, nv_a = state
        m = jnp.max(w, axis=1, keepdims=True)
        valid = m > NEG_INF
        eq = w == m
        besti = jnp.min(jnp.where(eq, idx_arr, NBOX), axis=1, keepdims=True)
        bm = eq & (idx_arr == besti)
        best = jnp.where(valid, besti, 0)

        def gthf(a, fallback):
            g = jnp.sum(jnp.where(bm, a, 0.0), axis=1, keepdims=True)
            return jnp.where(valid, g, fallback)

        by0 = gthf(y0, y00); bx0 = gthf(x0, x00)
        by1 = gthf(y1, y10); bx1 = gthf(x1, x10)
        bcid = jnp.sum(jnp.where(bm, cidv, 0), axis=1, keepdims=True)
        bcid = jnp.where(valid, bcid, cid0)
        bymin = jnp.minimum(by0, by1); bymax = jnp.maximum(by0, by1)
        bxmin = jnp.minimum(bx0, bx1); bxmax = jnp.maximum(bx0, bx1)
        inter_h = jnp.maximum(0.0, jnp.minimum(bymax, ymax)
                              - jnp.maximum(bymin, ymin))
        inter_w = jnp.maximum(0.0, jnp.minimum(bxmax, xmax)
                              - jnp.maximum(bxmin, xmin))
        inter = inter_h * inter_w
        area1 = (bymax - bymin) * (bxmax - bxmin)
        union = area1 + area - inter
        iou = jnp.where(union > 0, inter / union, 0.0)
        w_new = jnp.where(((iou > IOU_T) & valid) | bm, NEG_INF, w)

        hit = lane == i
        confp = jnp.where(valid, m, conf0)
        sel_a = jnp.where(hit, jnp.broadcast_to(best, acc_shape), sel_a)
        conf_a = jnp.where(hit, jnp.broadcast_to(confp, acc_shape), conf_a)
        cid_a = jnp.where(hit, jnp.broadcast_to(bcid, acc_shape), cid_a)
        b0_a = jnp.where(hit, jnp.broadcast_to(by0, acc_shape), b0_a)
        b1_a = jnp.where(hit, jnp.broadcast_to(bx0, acc_shape), b1_a)
        b2_a = jnp.where(hit, jnp.broadcast_to(by1, acc_shape), b2_a)
        b3_a = jnp.where(hit, jnp.broadcast_to(bx1, acc_shape), b3_a)
        nv_a = nv_a + jnp.broadcast_to(valid.astype(jnp.int32), acc_shape)
        return (sel_a, conf_a, cid_a, b0_a, b1_a, b2_a, b3_a, nv_a), w_new

    idxc = idxc_ref[...]
    y0c = y0c_ref[...]; x0c = x0c_ref[...]
    y1c = y1c_ref[...]; x1c = x1c_ref[...]
    cidc = cidc_ref[...]
    yminc = jnp.minimum(y0c, y1c); ymaxc = jnp.maximum(y0c, y1c)
    xminc = jnp.minimum(x0c, x1c); xmaxc = jnp.maximum(x0c, x1c)
    areac = (ymaxc - yminc) * (xmaxc - xminc)

    def tier_body(i, st):
        w, state = st
        state, w = round_core(w, idxc, y0c, x0c, y1c, x1c, cidc,
                              yminc, ymaxc, xminc, xmaxc, areac, i, state)
        return w, state

    w0 = sc_ref[...]
    _, tier_state = lax.fori_loop(0, MAXDET, tier_body, (w0, init))
    nv_tier = tier_state[7][:, 0:1]

    conf = conf_ref[...]
    c_all = jnp.sum((conf >= SCORE_T).astype(jnp.int32), axis=1,
                    keepdims=True)
    c_tier = jnp.sum((conf >= T_FIX).astype(jnp.int32), axis=1,
                     keepdims=True)
    below = c_all > c_tier
    over = jnp.max(wcnt_ref[...], axis=1, keepdims=True) > QUOTA
    need_full = over | (below & (nv_tier < MAXDET))
    any_full = jnp.max(need_full.astype(jnp.int32))

    def full_path(_):
        cf = conf_ref[...]
        work_ref[...] = jnp.where(cf >= SCORE_T, cf, NEG_INF)
        iota = lax.broadcasted_iota(jnp.int32, (NIMG, NBOX), 1)
        y0 = bx_ref[:, 0, :]; x0 = bx_ref[:, 1, :]
        y1 = bx_ref[:, 2, :]; x1 = bx_ref[:, 3, :]
        cidv = cid_ref[...]
        ymin = jnp.minimum(y0, y1); ymax = jnp.maximum(y0, y1)
        xmin = jnp.minimum(x0, x1); xmax = jnp.maximum(x0, x1)
        area = (ymax - ymin) * (xmax - xmin)

        def body(i, state):
            w = work_ref[...]
            state, w_new = round_core(w, iota, y0, x0, y1, x1, cidv,
                                      ymin, ymax, xmin, xmax, area, i, state)
            work_ref[...] = w_new
            return state

        return lax.fori_loop(0, MAXDET, body, init)

    def tier_path(_):
        return tier_state

    sel_a, conf_a, cid_a, b0_a, b1_a, b2_a, b3_a, nv_a = lax.cond(
        any_full > 0, full_path, tier_path, 0)
    selo_ref[...] = sel_a[:, :MAXDET]
    confo_ref[...] = conf_a[:, :MAXDET]
    cido_ref[...] = cid_a[:, :MAXDET]
    by0o_ref[...] = b0_a[:, :MAXDET]
    bx0o_ref[...] = b1_a[:, :MAXDET]
    by1o_ref[...] = b2_a[:, :MAXDET]
    bx1o_ref[...] = b3_a[:, :MAXDET]
    nvo_ref[...] = nv_a[:, :1]


def _run_conf(classes):
    nimg, n, nc = classes.shape
    cls_t = jnp.transpose(classes, (0, 2, 1))
    blk = 2048
    conf, cid = pl.pallas_call(
        _conf_kernel,
        grid=(pl.cdiv(n, blk),),
        in_specs=[pl.BlockSpec((nimg, nc, blk), lambda i: (0, 0, i))],
        out_specs=[pl.BlockSpec((nimg, blk), lambda i: (0, i)),
                   pl.BlockSpec((nimg, blk), lambda i: (0, i))],
        out_shape=[jax.ShapeDtypeStruct((nimg, n), jnp.float32),
                   jax.ShapeDtypeStruct((nimg, n), jnp.int32)],
    )(cls_t)
    return conf, cid


def _run_compact(conf, bx_t, cid):
    BUF = CHUNK + 16
    f32 = jnp.float32
    i32 = jnp.int32
    outs = pl.kernel(
        _sc_compact_kernel,
        out_type=[jax.ShapeDtypeStruct((NIMG * TIER,), f32),
                  jax.ShapeDtypeStruct((NIMG * TIER,), f32),
                  jax.ShapeDtypeStruct((NIMG * TIER,), f32),
                  jax.ShapeDtypeStruct((NIMG * TIER,), f32),
                  jax.ShapeDtypeStruct((NIMG * TIER,), f32),
                  jax.ShapeDtypeStruct((NIMG * TIER,), i32),
                  jax.ShapeDtypeStruct((NIMG * TIER,), i32),
                  jax.ShapeDtypeStruct((NWORK * 16,), i32)],
        mesh=plsc.VectorSubcoreMesh(core_axis_name="c", subcore_axis_name="s"),
        compiler_params=pltpu.CompilerParams(needs_layout_passes=False),
        scratch_types=[pltpu.VMEM((BUF,), f32),
                       pltpu.VMEM((BUF,), f32),
                       pltpu.VMEM((BUF,), f32),
                       pltpu.VMEM((BUF,), f32),
                       pltpu.VMEM((BUF,), f32),
                       pltpu.VMEM((BUF,), i32),
                       pltpu.VMEM((BUF + 16,), f32),
                       pltpu.VMEM((BUF + 16,), i32),
                       pltpu.VMEM((QUOTA,), f32),
                       pltpu.VMEM((QUOTA,), f32),
                       pltpu.VMEM((QUOTA,), f32),
                       pltpu.VMEM((QUOTA,), f32),
                       pltpu.VMEM((QUOTA,), i32),
                       pltpu.VMEM((QUOTA,), i32),
                       pltpu.VMEM((16,), i32)],
    )(conf.reshape(-1), bx_t.reshape(-1), cid.reshape(-1))
    sco, y0o, x0o, y1o, x1o, cido, idxo, cnts = outs
    shp = (NIMG, TIER)
    return (sco.reshape(shp), y0o.reshape(shp), x0o.reshape(shp),
            y1o.reshape(shp), x1o.reshape(shp), cido.reshape(shp),
            idxo.reshape(shp), cnts.reshape(NWORK, 16))


def kernel(boxes, classes):
    conf, cid = _run_conf(classes)
    bx_t = jnp.transpose(boxes, (0, 2, 1))
    sc, y0c, x0c, y1c, x1c, cidc, idxc, cnts = _run_compact(conf, bx_t, cid)
    wcnt = cnts.reshape(NIMG, WPI * 16)

    outs = pl.pallas_call(
        _nms_kernel,
        out_shape=[jax.ShapeDtypeStruct((NIMG, MAXDET), jnp.int32),
                   jax.ShapeDtypeStruct((NIMG, MAXDET), jnp.float32),
                   jax.ShapeDtypeStruct((NIMG, MAXDET), jnp.int32),
                   jax.ShapeDtypeStruct((NIMG, MAXDET), jnp.float32),
                   jax.ShapeDtypeStruct((NIMG, MAXDET), jnp.float32),
                   jax.ShapeDtypeStruct((NIMG, MAXDET), jnp.float32),
                   jax.ShapeDtypeStruct((NIMG, MAXDET), jnp.float32),
                   jax.ShapeDtypeStruct((NIMG, 1), jnp.int32)],
        scratch_shapes=[pltpu.VMEM((NIMG, NBOX), jnp.float32)],
    )(conf, bx_t, cid,
      sc, y0c, x0c, y1c, x1c, cidc, idxc, wcnt)
    sel, confp, cidp, by0, bx0, by1, bx1, nv = outs
    box_prediction = jnp.stack([by0, bx0, by1, bx1], axis=-1)
    return box_prediction, confp, cidp, nv[:, 0]


# speedup vs baseline: 2.7190x; 1.0003x over previous
import jax
import jax.numpy as jnp
from jax import lax
from jax.experimental import pallas as pl
from jax.experimental.pallas import tpu as pltpu
from jax.experimental.pallas import tpu_sc as plsc

IOU_T = 0.5
SCORE_T = 0.5
MAXDET = 100
NEG_INF = float("-inf")

NIMG = 8
NBOX = 20000
NCLS = 80
NWORK = 32
WPI = NWORK // NIMG
CHUNK = NBOX // WPI
QUOTA = 384
TIER = WPI * QUOTA
TIER_TOTAL = 1024
T_FIX = float((1.0 - TIER_TOTAL / NBOX) ** (1.0 / NCLS))


def _conf_kernel(cls_ref, conf_ref, cid_ref):
    c = cls_ref[...]
    m = jnp.max(c, axis=1)
    conf_ref[...] = m
    i3 = lax.broadcasted_iota(jnp.int32, c.shape, 1)
    cid_ref[...] = jnp.min(jnp.where(c == m[:, None, :], i3, c.shape[1]),
                           axis=1)


def _sc_compact_kernel(conf_hbm, box_hbm, cid_hbm,
                       sco_hbm, y0o_hbm, x0o_hbm, y1o_hbm, x1o_hbm,
                       cido_hbm, idxo_hbm, cnt_hbm,
                       conf_v, y0_v, x0_v, y1_v, x1_v, cid_v, sbuf, ibuf,
                       y0b, x0b, y1b, x1b, cidb, igb, cbuf):
    wid = lax.axis_index("s") * 2 + lax.axis_index("c")
    img = wid // WPI
    wq = wid % WPI
    base = img * NBOX + wq * CHUNK
    bbase = img * (NBOX * 4) + wq * CHUNK

    pltpu.sync_copy(conf_hbm.at[pl.ds(base, CHUNK)], conf_v.at[pl.ds(0, CHUNK)])
    pltpu.sync_copy(box_hbm.at[pl.ds(bbase, CHUNK)], y0_v.at[pl.ds(0, CHUNK)])
    pltpu.sync_copy(box_hbm.at[pl.ds(bbase + NBOX, CHUNK)],
                    x0_v.at[pl.ds(0, CHUNK)])
    pltpu.sync_copy(box_hbm.at[pl.ds(bbase + 2 * NBOX, CHUNK)],
                    y1_v.at[pl.ds(0, CHUNK)])
    pltpu.sync_copy(box_hbm.at[pl.ds(bbase + 3 * NBOX, CHUNK)],
                    x1_v.at[pl.ds(0, CHUNK)])
    pltpu.sync_copy(cid_hbm.at[pl.ds(base, CHUNK)], cid_v.at[pl.ds(0, CHUNK)])

    lane = lax.broadcasted_iota(jnp.int32, (16,), 0)

    def pre(j, _):
        sbuf[pl.ds(j * 16, 16)] = jnp.full((16,), NEG_INF, jnp.float32)
        ibuf[pl.ds(j * 16, 16)] = jnp.zeros((16,), jnp.int32)
        return 0
    lax.fori_loop(0, (QUOTA + 16) // 16, pre, 0)

    def scan(i, off):
        b = i * 16
        v = conf_v[pl.ds(b, 16)]
        li = lane + b
        m = (li < CHUNK) & (v >= T_FIX)
        incl = plsc.cumsum(m.astype(jnp.int32))
        dest = off + incl - 1
        plsc.store_scatter(sbuf, [dest], v, mask=m)
        plsc.store_scatter(ibuf, [dest], li, mask=m)
        return off + jnp.max(incl)
    nscan = (CHUNK + 15) // 16
    cnt = lax.fori_loop(0, nscan, scan, jnp.int32(0))

    def clean(j, _):
        b = j * 16
        sl = lane + b
        s = sbuf[pl.ds(b, 16)]
        iv = ibuf[pl.ds(b, 16)]
        sbuf[pl.ds(b, 16)] = jnp.where(sl < cnt, s, NEG_INF)
        ibuf[pl.ds(b, 16)] = jnp.where(sl < cnt, iv, 0)
        return 0
    lax.fori_loop(0, QUOTA // 16, clean, 0)

    def gath(j, _):
        b = j * 16
        il = ibuf[pl.ds(b, 16)]
        y0b[pl.ds(b, 16)] = plsc.load_gather(y0_v, [il])
        x0b[pl.ds(b, 16)] = plsc.load_gather(x0_v, [il])
        y1b[pl.ds(b, 16)] = plsc.load_gather(y1_v, [il])
        x1b[pl.ds(b, 16)] = plsc.load_gather(x1_v, [il])
        cidb[pl.ds(b, 16)] = plsc.load_gather(cid_v, [il])
        igb[pl.ds(b, 16)] = il + wq * CHUNK
        return 0
    lax.fori_loop(0, QUOTA // 16, gath, 0)

    obase = img * TIER + wq * QUOTA
    pltpu.sync_copy(sbuf.at[pl.ds(0, QUOTA)], sco_hbm.at[pl.ds(obase, QUOTA)])
    pltpu.sync_copy(y0b, y0o_hbm.at[pl.ds(obase, QUOTA)])
    pltpu.sync_copy(x0b, x0o_hbm.at[pl.ds(obase, QUOTA)])
    pltpu.sync_copy(y1b, y1o_hbm.at[pl.ds(obase, QUOTA)])
    pltpu.sync_copy(x1b, x1o_hbm.at[pl.ds(obase, QUOTA)])
    pltpu.sync_copy(cidb, cido_hbm.at[pl.ds(obase, QUOTA)])
    pltpu.sync_copy(igb, idxo_hbm.at[pl.ds(obase, QUOTA)])
    cbuf[...] = jnp.zeros((16,), jnp.int32) + cnt
    pltpu.sync_copy(cbuf, cnt_hbm.at[pl.ds(wid * 16, 16)])


def _nms_kernel(conf_ref, bx_ref, cid_ref,
                sc_ref, y0c_ref, x0c_ref, y1c_ref, x1c_ref,
                cidc_ref, idxc_ref, wcnt_ref,
                selo_ref, confo_ref, cido_ref,
                by0o_ref, bx0o_ref, by1o_ref, bx1o_ref, nvo_ref,
                work_ref):
    acc_shape = (NIMG, 128)
    lane = lax.broadcasted_iota(jnp.int32, acc_shape, 1)
    conf0 = conf_ref[:, 0:1]
    b00 = bx_ref[:, :, 0:1]
    y00 = b00[:, 0]; x00 = b00[:, 1]
    y10 = b00[:, 2]; x10 = b00[:, 3]
    cid0 = cid_ref[:, 0:1]
    zf = jnp.zeros(acc_shape, jnp.float32)
    zi = jnp.zeros(acc_shape, jnp.int32)
    init = (zi, zf, zi, zf, zf, zf, zf, zi)

    def round_core(w, idx_arr, y0, x0, y1, x1, cidv,
                   ymin, ymax, xmin, xmax, area, i, state):
        sel_a, conf_a, cid_a, b0_a, b1_a, b2_a, b3_a, nv_a = state
        m = jnp.max(w, axis=1, keepdims=True)
        valid = m > NEG_INF
        eq = w == m
        besti = jnp.min(jnp.where(eq, idx_arr, NBOX), axis=1, keepdims=True)
        bm = eq & (idx_arr == besti)
        best = jnp.where(valid, besti, 0)

        def gthf(a, fallback):
            g = jnp.sum(jnp.where(bm, a, 0.0), axis=1, keepdims=True)
            return jnp.where(valid, g, fallback)

        by0 = gthf(y0, y00); bx0 = gthf(x0, x00)
        by1 = gthf(y1, y10); bx1 = gthf(x1, x10)
        bcid = jnp.sum(jnp.where(bm, cidv, 0), axis=1, keepdims=True)
        bcid = jnp.where(valid, bcid, cid0)
        bymin = jnp.minimum(by0, by1); bymax = jnp.maximum(by0, by1)
        bxmin = jnp.minimum(bx0, bx1); bxmax = jnp.maximum(bx0, bx1)
        inter_h = jnp.maximum(0.0, jnp.minimum(bymax, ymax)
                              - jnp.maximum(bymin, ymin))
        inter_w = jnp.maximum(0.0, jnp.minimum(bxmax, xmax)
                              - jnp.maximum(bxmin, xmin))
        inter = inter_h * inter_w
        area1 = (bymax - bymin) * (bxmax - bxmin)
        union = area1 + area - inter
        iou = jnp.where(union > 0, inter / union, 0.0)
        w_new = jnp.where(((iou > IOU_T) & valid) | bm, NEG_INF, w)

        hit = lane == i
        confp = jnp.where(valid, m, conf0)
        sel_a = jnp.where(hit, jnp.broadcast_to(best, acc_shape), sel_a)
        conf_a = jnp.where(hit, jnp.broadcast_to(confp, acc_shape), conf_a)
        cid_a = jnp.where(hit, jnp.broadcast_to(bcid, acc_shape), cid_a)
        b0_a = jnp.where(hit, jnp.broadcast_to(by0, acc_shape), b0_a)
        b1_a = jnp.where(hit, jnp.broadcast_to(bx0, acc_shape), b1_a)
        b2_a = jnp.where(hit, jnp.broadcast_to(by1, acc_shape), b2_a)
        b3_a = jnp.where(hit, jnp.broadcast_to(bx1, acc_shape), b3_a)
        nv_a = nv_a + jnp.broadcast_to(valid.astype(jnp.int32), acc_shape)
        return (sel_a, conf_a, cid_a, b0_a, b1_a, b2_a, b3_a, nv_a), w_new

    idxc = idxc_ref[...]
    y0c = y0c_ref[...]; x0c = x0c_ref[...]
    y1c = y1c_ref[...]; x1c = x1c_ref[...]
    cidc = cidc_ref[...]
    yminc = jnp.minimum(y0c, y1c); ymaxc = jnp.maximum(y0c, y1c)
    xminc = jnp.minimum(x0c, x1c); xmaxc = jnp.maximum(x0c, x1c)
    areac = (ymaxc - yminc) * (xmaxc - xminc)

    def tier_body(i, st):
        w, state = st
        state, w = round_core(w, idxc, y0c, x0c, y1c, x1c, cidc,
                              yminc, ymaxc, xminc, xmaxc, areac, i, state)
        return w, state

    w0 = sc_ref[...]
    _, tier_state = lax.fori_loop(0, MAXDET, tier_body, (w0, init))
    nv_tier = tier_state[7][:, 0:1]

    conf = conf_ref[...]
    c_all = jnp.sum((conf >= SCORE_T).astype(jnp.int32), axis=1,
                    keepdims=True)
    c_tier = jnp.sum((conf >= T_FIX).astype(jnp.int32), axis=1,
                     keepdims=True)
    below = c_all > c_tier
    over = jnp.max(wcnt_ref[...], axis=1, keepdims=True) > QUOTA
    need_full = over | (below & (nv_tier < MAXDET))
    any_full = jnp.max(need_full.astype(jnp.int32))

    def full_path(_):
        cf = conf_ref[...]
        work_ref[...] = jnp.where(cf >= SCORE_T, cf, NEG_INF)
        iota = lax.broadcasted_iota(jnp.int32, (NIMG, NBOX), 1)
        y0 = bx_ref[:, 0, :]; x0 = bx_ref[:, 1, :]
        y1 = bx_ref[:, 2, :]; x1 = bx_ref[:, 3, :]
        cidv = cid_ref[...]
        ymin = jnp.minimum(y0, y1); ymax = jnp.maximum(y0, y1)
        xmin = jnp.minimum(x0, x1); xmax = jnp.maximum(x0, x1)
        area = (ymax - ymin) * (xmax - xmin)

        def body(i, state):
            w = work_ref[...]
            state, w_new = round_core(w, iota, y0, x0, y1, x1, cidv,
                                      ymin, ymax, xmin, xmax, area, i, state)
            work_ref[...] = w_new
            return state

        return lax.fori_loop(0, MAXDET, body, init)

    def tier_path(_):
        return tier_state

    sel_a, conf_a, cid_a, b0_a, b1_a, b2_a, b3_a, nv_a = lax.cond(
        any_full > 0, full_path, tier_path, 0)
    selo_ref[...] = sel_a[:, :MAXDET]
    confo_ref[...] = conf_a[:, :MAXDET]
    cido_ref[...] = cid_a[:, :MAXDET]
    by0o_ref[...] = b0_a[:, :MAXDET]
    bx0o_ref[...] = b1_a[:, :MAXDET]
    by1o_ref[...] = b2_a[:, :MAXDET]
    bx1o_ref[...] = b3_a[:, :MAXDET]
    nvo_ref[...] = nv_a[:, :1]


def _run_conf(classes):
    nimg, n, nc = classes.shape
    cls_t = jnp.transpose(classes, (0, 2, 1))
    blk = 2048
    conf, cid = pl.pallas_call(
        _conf_kernel,
        grid=(pl.cdiv(n, blk),),
        in_specs=[pl.BlockSpec((nimg, nc, blk), lambda i: (0, 0, i))],
        out_specs=[pl.BlockSpec((nimg, blk), lambda i: (0, i)),
                   pl.BlockSpec((nimg, blk), lambda i: (0, i))],
        out_shape=[jax.ShapeDtypeStruct((nimg, n), jnp.float32),
                   jax.ShapeDtypeStruct((nimg, n), jnp.int32)],
    )(cls_t)
    return conf, cid


def _run_compact(conf, bx_t, cid):
    BUF = CHUNK + 16
    f32 = jnp.float32
    i32 = jnp.int32
    outs = pl.kernel(
        _sc_compact_kernel,
        out_type=[jax.ShapeDtypeStruct((NIMG * TIER,), f32),
                  jax.ShapeDtypeStruct((NIMG * TIER,), f32),
                  jax.ShapeDtypeStruct((NIMG * TIER,), f32),
                  jax.ShapeDtypeStruct((NIMG * TIER,), f32),
                  jax.ShapeDtypeStruct((NIMG * TIER,), f32),
                  jax.ShapeDtypeStruct((NIMG * TIER,), i32),
                  jax.ShapeDtypeStruct((NIMG * TIER,), i32),
                  jax.ShapeDtypeStruct((NWORK * 16,), i32)],
        mesh=plsc.VectorSubcoreMesh(core_axis_name="c", subcore_axis_name="s"),
        compiler_params=pltpu.CompilerParams(needs_layout_passes=False),
        scratch_types=[pltpu.VMEM((BUF,), f32),
                       pltpu.VMEM((BUF,), f32),
                       pltpu.VMEM((BUF,), f32),
                       pltpu.VMEM((BUF,), f32),
                       pltpu.VMEM((BUF,), f32),
                       pltpu.VMEM((BUF,), i32),
                       pltpu.VMEM((BUF + 16,), f32),
                       pltpu.VMEM((BUF + 16,), i32),
                       pltpu.VMEM((QUOTA,), f32),
                       pltpu.VMEM((QUOTA,), f32),
                       pltpu.VMEM((QUOTA,), f32),
                       pltpu.VMEM((QUOTA,), f32),
                       pltpu.VMEM((QUOTA,), i32),
                       pltpu.VMEM((QUOTA,), i32),
                       pltpu.VMEM((16,), i32)],
    )(conf.reshape(-1), bx_t.reshape(-1), cid.reshape(-1))
    sco, y0o, x0o, y1o, x1o, cido, idxo, cnts = outs
    shp = (NIMG, TIER)
    return (sco.reshape(shp), y0o.reshape(shp), x0o.reshape(shp),
            y1o.reshape(shp), x1o.reshape(shp), cido.reshape(shp),
            idxo.reshape(shp), cnts.reshape(NWORK, 16))


def kernel(boxes, classes):
    conf, cid = _run_conf(classes)
    bx_t = jnp.transpose(boxes, (0, 2, 1))
    sc, y0c, x0c, y1c, x1c, cidc, idxc, cnts = _run_compact(conf, bx_t, cid)
    wcnt = cnts.reshape(NIMG, WPI * 16)

    outs = pl.pallas_call(
        _nms_kernel,
        out_shape=[jax.ShapeDtypeStruct((NIMG, MAXDET), jnp.int32),
                   jax.ShapeDtypeStruct((NIMG, MAXDET), jnp.float32),
                   jax.ShapeDtypeStruct((NIMG, MAXDET), jnp.int32),
                   jax.ShapeDtypeStruct((NIMG, MAXDET), jnp.float32),
                   jax.ShapeDtypeStruct((NIMG, MAXDET), jnp.float32),
                   jax.ShapeDtypeStruct((NIMG, MAXDET), jnp.float32),
                   jax.ShapeDtypeStruct((NIMG, MAXDET), jnp.float32),
                   jax.ShapeDtypeStruct((NIMG, 1), jnp.int32)],
        scratch_shapes=[pltpu.VMEM((NIMG, NBOX), jnp.float32)],
    )(conf, bx_t, cid,
      sc, y0c, x0c, y1c, x1c, cidc, idxc, wcnt)
    sel, confp, cidp, by0, bx0, by1, bx1, nv = outs
    box_prediction = jnp.stack([by0, bx0, by1, bx1], axis=-1)
    return box_prediction, confp, cidp, nv[:, 0]
